# baseline (speedup 1.0000x reference)
import functools

import jax
import jax.numpy as jnp
from jax import lax
from jax.experimental import pallas as pl
from jax.experimental.pallas import tpu as pltpu
from jax.experimental.pallas import tpu_sc as plsc

_S = 30.0


def _tc_body(logits_ref, target_ref, m_ref, out_ref, *, rows, n_classes):
    i = pl.program_id(0)

    x = logits_ref[...]
    t = target_ref[0, 0, :]
    m = m_ref[0, :]

    lane = lax.broadcasted_iota(jnp.int32, (rows, n_classes), 1)
    tmask = (lane == t[:, None]).astype(jnp.float32)

    ones_col = jnp.ones((n_classes, 1), jnp.float32)
    w2_col = 1.0 - jnp.exp(-_S * m).reshape(n_classes, 1)

    row_max = jnp.max(x, axis=1, keepdims=True)
    e = jnp.exp(x - row_max)

    l_t = jnp.dot(x * tmask, ones_col, preferred_element_type=jnp.float32)
    se_all = jnp.dot(e, ones_col, preferred_element_type=jnp.float32)
    corr = jnp.dot(e * tmask, w2_col, preferred_element_type=jnp.float32)

    lg = jnp.log(se_all - corr)
    partial = jnp.sum(row_max + lg - l_t, axis=(0, 1), keepdims=True)

    @pl.when(i == 0)
    def _():
        out_ref[...] = jnp.zeros_like(out_ref)

    out_ref[...] += partial


def _tc_pass(logits, m_list, target, rows):
    batch, n_classes = logits.shape
    grid = batch // rows
    target3 = target.reshape(grid, 1, rows)
    m2 = m_list.reshape(1, n_classes)
    body = functools.partial(_tc_body, rows=rows, n_classes=n_classes)
    return pl.pallas_call(
        body,
        grid=(grid,),
        in_specs=[
            pl.BlockSpec((rows, n_classes), lambda i: (i, 0)),
            pl.BlockSpec((1, 1, rows), lambda i: (i, 0, 0)),
            pl.BlockSpec((1, n_classes), lambda i: (0, 0)),
        ],
        out_specs=pl.BlockSpec((1, 1), lambda i: (0, 0)),
        out_shape=jax.ShapeDtypeStruct((1, 1), jnp.float32),
    )(logits, target3, m2)


def _sc_margin_partials(m_list, target):
    batch = target.shape[0]
    info = plsc.get_sparse_core_info()
    nc, ns, nl = info.num_cores, info.num_subcores, info.num_lanes
    nw = nc * ns
    bpw = batch // nw
    mesh = plsc.VectorSubcoreMesh(core_axis_name="c", subcore_axis_name="s")
    chunk = 128

    @functools.partial(
        pl.kernel,
        mesh=mesh,
        out_type=jax.ShapeDtypeStruct((nw, nl), jnp.float32),
        scratch_types=[
            pltpu.VMEM((bpw,), jnp.int32),
            pltpu.VMEM((bpw,), jnp.float32),
            pltpu.VMEM((nl,), jnp.float32),
            pltpu.SemaphoreType.DMA,
        ],
    )
    def k(m_hbm, tgt_hbm, out_hbm, tgt_v, vals_v, acc_v, sem):
        wid = lax.axis_index("s") * nc + lax.axis_index("c")
        base = wid * bpw
        pltpu.sync_copy(tgt_hbm.at[pl.ds(base, bpw)], tgt_v)
        for j in range(bpw // chunk):
            pltpu.async_copy(
                m_hbm.at[tgt_v.at[pl.ds(j * chunk, chunk)]],
                vals_v.at[pl.ds(j * chunk, chunk)],
                sem,
            ).wait()

        acc = jnp.zeros((nl,), jnp.float32)
        for j in range(bpw // nl):
            acc = acc + vals_v[pl.ds(j * nl, nl)]
        acc_v[...] = acc * _S
        pltpu.sync_copy(acc_v, out_hbm.at[wid])

    return k(m_list, target)


def kernel(logits, m_list, target):
    batch = logits.shape[0]
    sc_parts = _sc_margin_partials(m_list, target)
    tc_part = _tc_pass(logits, m_list, target, rows=4096)
    return (tc_part[0, 0] + jnp.sum(sc_parts)) * (1.0 / batch)


# speedup vs baseline: 5.0989x; 5.0989x over previous
import functools

import jax
import jax.numpy as jnp
from jax import lax
from jax.experimental import pallas as pl

_S = 30.0


def _ldam_body(logits_ref, target_ref, m_ref, out_ref, *, rows, n_classes, batch):
    i = pl.program_id(0)

    x = logits_ref[...]
    t = target_ref[0, 0, :]
    m = m_ref[0, :]

    lane = lax.broadcasted_iota(jnp.int32, (rows, n_classes), 1)
    tmask = (lane == t[:, None]).astype(jnp.float32)

    ones_col = jnp.ones((n_classes, 1), jnp.float32)
    m_col = m.reshape(n_classes, 1)

    row_max = jnp.max(x, axis=1, keepdims=True)
    e = jnp.exp(x - row_max)

    l_t = jnp.dot(x * tmask, ones_col, preferred_element_type=jnp.float32)
    m_row = jnp.dot(tmask, m_col, preferred_element_type=jnp.float32)
    se_all = jnp.dot(e, ones_col, preferred_element_type=jnp.float32)

    a = l_t - _S * m_row
    e_t = jnp.exp(l_t - row_max)
    se_adj = se_all - e_t + jnp.exp(a - row_max)
    nll = row_max + jnp.log(se_adj) - a

    partial = jnp.sum(nll, axis=(0, 1), keepdims=True) * (1.0 / batch)

    @pl.when(i == 0)
    def _():
        out_ref[...] = jnp.zeros_like(out_ref)

    out_ref[...] += partial


def kernel(logits, m_list, target):
    batch, n_classes = logits.shape
    rows = 4096
    grid = batch // rows

    target3 = target.reshape(grid, 1, rows)
    m2 = m_list.reshape(1, n_classes)

    body = functools.partial(_ldam_body, rows=rows, n_classes=n_classes, batch=batch)
    out = pl.pallas_call(
        body,
        grid=(grid,),
        in_specs=[
            pl.BlockSpec((rows, n_classes), lambda i: (i, 0)),
            pl.BlockSpec((1, 1, rows), lambda i: (i, 0, 0)),
            pl.BlockSpec((1, n_classes), lambda i: (0, 0)),
        ],
        out_specs=pl.BlockSpec((1, 1), lambda i: (0, 0)),
        out_shape=jax.ShapeDtypeStruct((1, 1), jnp.float32),
    )(logits, target3, m2)
    return out[0, 0]


# speedup vs baseline: 5.1143x; 1.0030x over previous
import functools

import jax
import jax.numpy as jnp
from jax import lax
from jax.experimental import pallas as pl

_S = 30.0


def _ldam_body(logits_ref, target_ref, m_ref, out_ref, *, rows, n_classes, batch):
    i = pl.program_id(0)

    x = logits_ref[...]
    t = target_ref[...]
    m = m_ref[0, :]

    lane = lax.broadcasted_iota(jnp.int32, (rows, n_classes), 1)
    tmask = (lane == t[:, None]).astype(jnp.float32)

    ones_col = jnp.ones((n_classes, 1), jnp.float32)
    m_col = m.reshape(n_classes, 1)

    row_max = jnp.max(x, axis=1, keepdims=True)
    e = jnp.exp(x - row_max)

    l_t = jnp.dot(x * tmask, ones_col, preferred_element_type=jnp.float32)
    m_row = jnp.dot(tmask, m_col, preferred_element_type=jnp.float32)
    se_all = jnp.dot(e, ones_col, preferred_element_type=jnp.float32)

    a = l_t - _S * m_row
    e_t = jnp.exp(l_t - row_max)
    se_adj = se_all - e_t + jnp.exp(a - row_max)
    nll = row_max + jnp.log(se_adj) - a

    partial = jnp.sum(nll, axis=(0, 1), keepdims=True) * (1.0 / batch)

    @pl.when(i == 0)
    def _():
        out_ref[...] = jnp.zeros_like(out_ref)

    out_ref[...] += partial


def kernel(logits, m_list, target):
    batch, n_classes = logits.shape
    rows = 4096
    grid = batch // rows

    m2 = m_list.reshape(1, n_classes)

    body = functools.partial(_ldam_body, rows=rows, n_classes=n_classes, batch=batch)
    out = pl.pallas_call(
        body,
        grid=(grid,),
        in_specs=[
            pl.BlockSpec((rows, n_classes), lambda i: (i, 0)),
            pl.BlockSpec((rows,), lambda i: (i,)),
            pl.BlockSpec((1, n_classes), lambda i: (0, 0)),
        ],
        out_specs=pl.BlockSpec((1, 1), lambda i: (0, 0)),
        out_shape=jax.ShapeDtypeStruct((1, 1), jnp.float32),
    )(logits, target, m2)
    return out[0, 0]
